# baseline (speedup 1.0000x reference)
import functools

import jax
import jax.numpy as jnp
from jax import lax
from jax.experimental import pallas as pl
from jax.experimental.pallas import tpu as pltpu
from jax.experimental.pallas import tpu_sc as plsc

_NB = 1000000
_D = 32
_DP = 32
_L = 16
_TI = 128
_F = 100
_FP = 103
_NBUF = 5


def _hash16(v):
    q = (v.astype(jnp.float32) * jnp.float32(1e-6)).astype(jnp.int32)
    r = v - q * _NB
    r = jnp.where(r < 0, r + _NB, r)
    r = jnp.where(r >= _NB, r - _NB, r)
    return r


def _make_gather(n_rows: int):
    info = plsc.get_sparse_core_info()
    nc, ns = info.num_cores, info.num_subcores
    nw = nc * ns
    n_tiles = n_rows // _TI
    t_per_w = n_tiles // nw
    assert t_per_w * nw == n_tiles and n_tiles * _TI == n_rows
    chunk = _TI * _FP

    mesh = plsc.VectorSubcoreMesh(core_axis_name="c", subcore_axis_name="s")

    @functools.partial(
        pl.kernel,
        mesh=mesh,
        compiler_params=pltpu.CompilerParams(use_tc_tiling_on_sc=False,
                                             needs_layout_passes=False),
        out_type=jax.ShapeDtypeStruct((n_rows * _F * _D,), jnp.float32),
        scratch_types=(
            [pltpu.VMEM((chunk,), jnp.int32),
             pltpu.VMEM((_F * _TI,), jnp.int32)]
            + [pltpu.VMEM((_TI, _DP), jnp.float32) for _ in range(_NBUF)]
            + [pltpu.VMEM((_D * _TI,), jnp.float32) for _ in range(_NBUF)]
            + [pltpu.SemaphoreType.DMA for _ in range(2 * _NBUF)]
        ),
    )
    def k(x_hbm, table_hbm, out_hbm, idx_raw, idx_t, *bufs):
        rows_v = bufs[:_NBUF]
        tblk_v = bufs[_NBUF:2 * _NBUF]
        gsem = bufs[2 * _NBUF:3 * _NBUF]
        wsem = bufs[3 * _NBUF:4 * _NBUF]

        wid = lax.axis_index("s") * nc + lax.axis_index("c")
        lane_iota = lax.iota(jnp.int32, _L)

        def start_gather(j, b):
            pltpu.async_copy(
                table_hbm.at[idx_t.at[pl.ds(j * _TI, _TI)]],
                rows_v[b], gsem[b])

        def wait_gather(j, b):
            pltpu.make_async_copy(
                table_hbm.at[idx_t.at[pl.ds(j * _TI, _TI)]],
                rows_v[b], gsem[b]).wait()

        def transpose_block(b):
            lane128 = lane_iota * _TI

            def d_body(d, c2):
                rot = lax.bitwise_and(lane_iota + d, _L - 1)
                for h in range(_D // _L):
                    cvec = lane_iota + h * _L
                    for g in range(_TI // _L):
                        vals = plsc.load_gather(
                            rows_v[b], [rot + g * _L, cvec])
                        didx = lane128 + (h * _L * _TI + g * _L) + rot
                        plsc.store_scatter(tblk_v[b], [didx], vals)
                return c2

            lax.fori_loop(0, _L, d_body, 0, unroll=4)

        def start_write(j, t, b):
            for band in range(4):
                pltpu.async_copy(
                    tblk_v[b].at[pl.ds(band * 1024, 1024)],
                    out_hbm.at[pl.ds(((j * 4 + band) * n_tiles + t) * 1024,
                                     1024)],
                    wsem[b])

        def wait_write(j, t, b):
            for band in range(4):
                pltpu.make_async_copy(
                    tblk_v[b].at[pl.ds(band * 1024, 1024)],
                    out_hbm.at[pl.ds(((j * 4 + band) * n_tiles + t) * 1024,
                                     1024)],
                    wsem[b]).wait()

        def t_body(tt, carry):
            t = wid * t_per_w + tt
            pltpu.sync_copy(x_hbm.at[pl.ds(t * chunk, chunk)], idx_raw)

            def hash_body(i, c2):
                v = idx_raw[pl.ds(i * _L, _L)]
                idx_raw[pl.ds(i * _L, _L)] = _hash16(v)
                return c2

            lax.fori_loop(0, chunk // _L, hash_body, 0, unroll=4)

            def tr_body(gr, c2):
                j = gr // (_TI // _L)
                g = gr % (_TI // _L)
                src = lane_iota * _FP + (g * _L * _FP + j)
                vals = plsc.load_gather(idx_raw, [src])
                idx_t[pl.ds(j * _TI + g * _L, _L)] = vals
                return c2

            lax.fori_loop(0, _F * (_TI // _L), tr_body, 0, unroll=8)

            for b in range(_NBUF):
                start_gather(b, b)

            def j_outer(oo, c2):
                o = oo * _NBUF
                for b in range(_NBUF):
                    j = o + b
                    wait_gather(j, b)

                    @pl.when(oo > 0)
                    def _():
                        wait_write(j - _NBUF, t, b)

                    transpose_block(b)
                    start_write(j, t, b)

                    @pl.when(o < _F - _NBUF)
                    def _():
                        start_gather(j + _NBUF, b)

                return c2

            lax.fori_loop(0, _F // _NBUF, j_outer, 0, unroll=False)

            for b in range(_NBUF):
                wait_write(_F - _NBUF + b, t, b)
            return carry

        lax.fori_loop(0, t_per_w, t_body, 0, unroll=False)

    return k


def kernel(x, table):
    b, f = x.shape
    xp = jnp.pad(x.astype(jnp.int32), ((0, 0), (0, _FP - f))).reshape(-1)
    flat = _make_gather(b)(xp, table)
    n_tiles = b // _TI
    out5 = flat.reshape(f, 4, n_tiles, 8, _TI)
    return jnp.transpose(out5, (2, 4, 0, 1, 3)).reshape(b, f, _D)


# speedup vs baseline: 1.6452x; 1.6452x over previous
import functools

import jax
import jax.numpy as jnp
from jax import lax
from jax.experimental import pallas as pl
from jax.experimental.pallas import tpu as pltpu
from jax.experimental.pallas import tpu_sc as plsc

_NB = 1000000
_D = 32
_DP = 32
_L = 16
_TI = 128
_F = 100
_FP = 103
_NBUF = 4


def _hash16(v):
    q = (v.astype(jnp.float32) * jnp.float32(1e-6)).astype(jnp.int32)
    r = v - q * _NB
    r = jnp.where(r < 0, r + _NB, r)
    r = jnp.where(r >= _NB, r - _NB, r)
    return r


def _make_gather(n_rows: int):
    info = plsc.get_sparse_core_info()
    nc, ns = info.num_cores, info.num_subcores
    nw = nc * ns
    n_tiles = n_rows // _TI
    t_per_w = n_tiles // nw
    assert t_per_w * nw == n_tiles and n_tiles * _TI == n_rows
    chunk = _TI * _FP

    mesh = plsc.VectorSubcoreMesh(core_axis_name="c", subcore_axis_name="s")

    @functools.partial(
        pl.kernel,
        mesh=mesh,
        compiler_params=pltpu.CompilerParams(use_tc_tiling_on_sc=False,
                                             needs_layout_passes=False),
        out_type=jax.ShapeDtypeStruct((n_rows * _F * _D,), jnp.float32),
        scratch_types=(
            [pltpu.VMEM((chunk,), jnp.int32),
             pltpu.VMEM((_F * _TI,), jnp.int32)]
            + [pltpu.VMEM((_TI, _DP), jnp.float32) for _ in range(_NBUF)]
            + [pltpu.VMEM((_D * _TI,), jnp.float32) for _ in range(_NBUF)]
            + [pltpu.SemaphoreType.DMA for _ in range(2 * _NBUF)]
        ),
    )
    def k(x_hbm, table_hbm, out_hbm, idx_raw, idx_t, *bufs):
        rows_v = bufs[:_NBUF]
        tblk_v = bufs[_NBUF:2 * _NBUF]
        gsem = bufs[2 * _NBUF:3 * _NBUF]
        wsem = bufs[3 * _NBUF:4 * _NBUF]

        wid = lax.axis_index("s") * nc + lax.axis_index("c")
        lane_iota = lax.iota(jnp.int32, _L)

        def start_gather(j, b):
            pltpu.async_copy(
                table_hbm.at[idx_t.at[pl.ds(j * _TI, _TI)]],
                rows_v[b], gsem[b])

        def wait_gather(j, b):
            pltpu.make_async_copy(
                table_hbm.at[idx_t.at[pl.ds(j * _TI, _TI)]],
                rows_v[b], gsem[b]).wait()

        def transpose_block(b):
            lane128 = lane_iota * _TI

            def d_body(d, c2):
                rot = lax.bitwise_and(lane_iota + d, _L - 1)
                vals = []
                for h in range(_D // _L):
                    cvec = lane_iota + h * _L
                    for g in range(_TI // _L):
                        vals.append(plsc.load_gather(
                            rows_v[b], [rot + g * _L, cvec]))
                i = 0
                for h in range(_D // _L):
                    for g in range(_TI // _L):
                        didx = lane128 + (h * _L * _TI + g * _L) + rot
                        plsc.store_scatter(tblk_v[b], [didx], vals[i])
                        i += 1
                return c2

            lax.fori_loop(0, _L, d_body, 0, unroll=2)

        def start_write(j, t, b):
            for band in range(4):
                pltpu.async_copy(
                    tblk_v[b].at[pl.ds(band * 1024, 1024)],
                    out_hbm.at[pl.ds(((j * 4 + band) * n_tiles + t) * 1024,
                                     1024)],
                    wsem[b])

        def wait_write(j, t, b):
            for band in range(4):
                pltpu.make_async_copy(
                    tblk_v[b].at[pl.ds(band * 1024, 1024)],
                    out_hbm.at[pl.ds(((j * 4 + band) * n_tiles + t) * 1024,
                                     1024)],
                    wsem[b]).wait()

        def t_body(tt, carry):
            t = wid * t_per_w + tt
            pltpu.sync_copy(x_hbm.at[pl.ds(t * chunk, chunk)], idx_raw)

            def hash_body(i, c2):
                v = idx_raw[pl.ds(i * _L, _L)]
                idx_raw[pl.ds(i * _L, _L)] = _hash16(v)
                return c2

            lax.fori_loop(0, chunk // _L, hash_body, 0, unroll=4)

            def tr_body(gr, c2):
                j = gr // (_TI // _L)
                g = gr % (_TI // _L)
                src = lane_iota * _FP + (g * _L * _FP + j)
                vals = plsc.load_gather(idx_raw, [src])
                idx_t[pl.ds(j * _TI + g * _L, _L)] = vals
                return c2

            lax.fori_loop(0, _F * (_TI // _L), tr_body, 0, unroll=8)

            for b in range(_NBUF):
                start_gather(b, b)

            def j_outer(oo, c2):
                o = oo * _NBUF
                for b in range(_NBUF):
                    j = o + b
                    wait_gather(j, b)

                    @pl.when(oo > 0)
                    def _():
                        wait_write(j - _NBUF, t, b)

                    transpose_block(b)
                    start_write(j, t, b)

                    @pl.when(o < _F - _NBUF)
                    def _():
                        start_gather(j + _NBUF, b)

                return c2

            lax.fori_loop(0, _F // _NBUF, j_outer, 0, unroll=False)

            for b in range(_NBUF):
                wait_write(_F - _NBUF + b, t, b)
            return carry

        lax.fori_loop(0, t_per_w, t_body, 0, unroll=False)

    return k


def kernel(x, table):
    b, f = x.shape
    xp = jnp.pad(x.astype(jnp.int32), ((0, 0), (0, _FP - f))).reshape(-1)
    flat = _make_gather(b)(xp, table)
    n_tiles = b // _TI
    out5 = flat.reshape(f, 4, n_tiles, 8, _TI)
    return jnp.transpose(out5, (2, 4, 0, 1, 3)).reshape(b, f, _D)


# speedup vs baseline: 1.6537x; 1.0051x over previous
import functools

import jax
import jax.numpy as jnp
from jax import lax
from jax.experimental import pallas as pl
from jax.experimental.pallas import tpu as pltpu
from jax.experimental.pallas import tpu_sc as plsc

_NB = 1000000
_D = 32
_DP = 32
_L = 16
_TI = 128
_F = 100
_FP = 103
_NBUF = 4


def _hash16(v):
    q = (v.astype(jnp.float32) * jnp.float32(1e-6)).astype(jnp.int32)
    r = v - q * _NB
    r = jnp.where(r < 0, r + _NB, r)
    r = jnp.where(r >= _NB, r - _NB, r)
    return r


def _make_gather(n_rows: int):
    info = plsc.get_sparse_core_info()
    nc, ns = info.num_cores, info.num_subcores
    nw = nc * ns
    n_tiles = n_rows // _TI
    t_per_w = n_tiles // nw
    assert t_per_w * nw == n_tiles and n_tiles * _TI == n_rows
    chunk = _TI * _FP

    mesh = plsc.VectorSubcoreMesh(core_axis_name="c", subcore_axis_name="s")

    @functools.partial(
        pl.kernel,
        mesh=mesh,
        compiler_params=pltpu.CompilerParams(use_tc_tiling_on_sc=False,
                                             needs_layout_passes=False),
        out_type=jax.ShapeDtypeStruct((n_rows * _F * _D,), jnp.float32),
        scratch_types=(
            [pltpu.VMEM((chunk,), jnp.int32),
             pltpu.VMEM((_F * _TI,), jnp.int32)]
            + [pltpu.VMEM((_TI, _DP), jnp.float32) for _ in range(_NBUF)]
            + [pltpu.VMEM((_D * _TI,), jnp.float32) for _ in range(_NBUF)]
            + [pltpu.SemaphoreType.DMA for _ in range(2 * _NBUF)]
        ),
    )
    def k(x_hbm, table_hbm, out_hbm, idx_raw, idx_t, *bufs):
        rows_v = bufs[:_NBUF]
        tblk_v = bufs[_NBUF:2 * _NBUF]
        gsem = bufs[2 * _NBUF:3 * _NBUF]
        wsem = bufs[3 * _NBUF:4 * _NBUF]

        wid = lax.axis_index("s") * nc + lax.axis_index("c")
        lane_iota = lax.iota(jnp.int32, _L)

        def start_gather(j, b):
            pltpu.async_copy(
                table_hbm.at[idx_t.at[pl.ds(j * _TI, _TI)]],
                rows_v[b], gsem[b])

        def wait_gather(j, b):
            pltpu.make_async_copy(
                table_hbm.at[idx_t.at[pl.ds(j * _TI, _TI)]],
                rows_v[b], gsem[b]).wait()

        def transpose_block(b):
            lane128 = lane_iota * _TI

            def d_body(d, c2):
                rot = lax.bitwise_and(lane_iota + d, _L - 1)
                vals = []
                for h in range(_D // _L):
                    cvec = lane_iota + h * _L
                    for g in range(_TI // _L):
                        vals.append(plsc.load_gather(
                            rows_v[b], [rot + g * _L, cvec]))
                i = 0
                for h in range(_D // _L):
                    for g in range(_TI // _L):
                        didx = lane128 + (h * _L * _TI + g * _L) + rot
                        plsc.store_scatter(tblk_v[b], [didx], vals[i])
                        i += 1
                return c2

            lax.fori_loop(0, _L, d_body, 0, unroll=1)

        def start_write(j, t, b):
            for band in range(4):
                pltpu.async_copy(
                    tblk_v[b].at[pl.ds(band * 1024, 1024)],
                    out_hbm.at[pl.ds(((j * 4 + band) * n_tiles + t) * 1024,
                                     1024)],
                    wsem[b])

        def wait_write(j, t, b):
            for band in range(4):
                pltpu.make_async_copy(
                    tblk_v[b].at[pl.ds(band * 1024, 1024)],
                    out_hbm.at[pl.ds(((j * 4 + band) * n_tiles + t) * 1024,
                                     1024)],
                    wsem[b]).wait()

        def t_body(tt, carry):
            t = wid * t_per_w + tt
            pltpu.sync_copy(x_hbm.at[pl.ds(t * chunk, chunk)], idx_raw)

            def hash_body(i, c2):
                v = idx_raw[pl.ds(i * _L, _L)]
                idx_raw[pl.ds(i * _L, _L)] = _hash16(v)
                return c2

            lax.fori_loop(0, chunk // _L, hash_body, 0, unroll=4)

            def tr_body(gr, c2):
                j = gr // (_TI // _L)
                g = gr % (_TI // _L)
                src = lane_iota * _FP + (g * _L * _FP + j)
                vals = plsc.load_gather(idx_raw, [src])
                idx_t[pl.ds(j * _TI + g * _L, _L)] = vals
                return c2

            lax.fori_loop(0, _F * (_TI // _L), tr_body, 0, unroll=8)

            for b in range(_NBUF):
                start_gather(b, b)

            def j_outer(oo, c2):
                o = oo * _NBUF
                for b in range(_NBUF):
                    j = o + b
                    wait_gather(j, b)

                    @pl.when(oo > 0)
                    def _():
                        wait_write(j - _NBUF, t, b)

                    transpose_block(b)
                    start_write(j, t, b)

                    @pl.when(o < _F - _NBUF)
                    def _():
                        start_gather(j + _NBUF, b)

                return c2

            lax.fori_loop(0, _F // _NBUF, j_outer, 0, unroll=False)

            for b in range(_NBUF):
                wait_write(_F - _NBUF + b, t, b)
            return carry

        lax.fori_loop(0, t_per_w, t_body, 0, unroll=False)

    return k


def kernel(x, table):
    b, f = x.shape
    xp = jnp.pad(x.astype(jnp.int32), ((0, 0), (0, _FP - f))).reshape(-1)
    flat = _make_gather(b)(xp, table)
    n_tiles = b // _TI
    out5 = flat.reshape(f, 4, n_tiles, 8, _TI)
    return jnp.transpose(out5, (2, 4, 0, 1, 3)).reshape(b, f, _D)


# speedup vs baseline: 3.5655x; 2.1561x over previous
import functools

import jax
import jax.numpy as jnp
from jax import lax
from jax.experimental import pallas as pl
from jax.experimental.pallas import tpu as pltpu
from jax.experimental.pallas import tpu_sc as plsc

_NB = 1000000
_D = 32
_DP = 32
_L = 16
_TI = 128
_F = 100
_FP = 103
_NBUF = 4


def _hash16(v):
    q = (v.astype(jnp.float32) * jnp.float32(1e-6)).astype(jnp.int32)
    r = v - q * _NB
    r = jnp.where(r < 0, r + _NB, r)
    r = jnp.where(r >= _NB, r - _NB, r)
    return r


def _make_gather(n_rows: int):
    info = plsc.get_sparse_core_info()
    nc, ns = info.num_cores, info.num_subcores
    nw = nc * ns
    n_tiles = n_rows // _TI
    t_per_w = n_tiles // nw
    assert t_per_w * nw == n_tiles and n_tiles * _TI == n_rows
    chunk = _TI * _FP

    mesh = plsc.VectorSubcoreMesh(core_axis_name="c", subcore_axis_name="s")

    @functools.partial(
        pl.kernel,
        mesh=mesh,
        compiler_params=pltpu.CompilerParams(use_tc_tiling_on_sc=False,
                                             needs_layout_passes=False),
        out_type=jax.ShapeDtypeStruct((n_rows * _F * _D,), jnp.float32),
        scratch_types=(
            [pltpu.VMEM((chunk,), jnp.int32),
             pltpu.VMEM((_F * _TI,), jnp.int32)]
            + [pltpu.VMEM((_TI, _DP), jnp.float32) for _ in range(_NBUF)]
            + [pltpu.VMEM((_D * _TI,), jnp.float32) for _ in range(_NBUF)]
            + [pltpu.SemaphoreType.DMA for _ in range(2 * _NBUF)]
        ),
    )
    def k(x_hbm, table_hbm, out_hbm, idx_raw, idx_t, *bufs):
        rows_v = bufs[:_NBUF]
        tblk_v = bufs[_NBUF:2 * _NBUF]
        gsem = bufs[2 * _NBUF:3 * _NBUF]
        wsem = bufs[3 * _NBUF:4 * _NBUF]

        wid = lax.axis_index("s") * nc + lax.axis_index("c")
        lane_iota = lax.iota(jnp.int32, _L)

        def start_gather(j, b):
            pltpu.async_copy(
                table_hbm.at[idx_t.at[pl.ds(j * _TI, _TI)]],
                rows_v[b], gsem[b])

        def wait_gather(j, b):
            pltpu.make_async_copy(
                table_hbm.at[idx_t.at[pl.ds(j * _TI, _TI)]],
                rows_v[b], gsem[b]).wait()

        def transpose_block(b):
            lane128 = lane_iota * _TI

            def d_body(d, c2):
                rot = lax.bitwise_and(lane_iota + d, _L - 1)
                vals = []
                for h in range(_D // _L):
                    cvec = lane_iota + h * _L
                    for g in range(_TI // _L):
                        vals.append(plsc.load_gather(
                            rows_v[b], [rot + g * _L, cvec]))
                i = 0
                for h in range(_D // _L):
                    for g in range(_TI // _L):
                        didx = lane128 + (h * _L * _TI + g * _L) + rot
                        plsc.store_scatter(tblk_v[b], [didx], vals[i])
                        i += 1
                return c2

            lax.fori_loop(0, _L, d_body, 0, unroll=1)

        def start_write(j, t, b):
            for band in range(4):
                pltpu.async_copy(
                    tblk_v[b].at[pl.ds(band * 1024, 1024)],
                    out_hbm.at[pl.ds(((j * 4 + band) * n_tiles + t) * 1024,
                                     1024)],
                    wsem[b])

        def wait_write(j, t, b):
            for band in range(4):
                pltpu.make_async_copy(
                    tblk_v[b].at[pl.ds(band * 1024, 1024)],
                    out_hbm.at[pl.ds(((j * 4 + band) * n_tiles + t) * 1024,
                                     1024)],
                    wsem[b]).wait()

        def t_body(tt, carry):
            t = wid * t_per_w + tt
            pltpu.sync_copy(x_hbm.at[pl.ds(t * chunk, chunk)], idx_raw)

            def hash_body(i, c2):
                v = idx_raw[pl.ds(i * _L, _L)]
                idx_raw[pl.ds(i * _L, _L)] = _hash16(v)
                return c2

            lax.fori_loop(0, chunk // _L, hash_body, 0, unroll=4)

            def tr_body(gr, c2):
                j = gr // (_TI // _L)
                g = gr % (_TI // _L)
                src = lane_iota * _FP + (g * _L * _FP + j)
                vals = plsc.load_gather(idx_raw, [src])
                idx_t[pl.ds(j * _TI + g * _L, _L)] = vals
                return c2

            lax.fori_loop(0, _F * (_TI // _L), tr_body, 0, unroll=8)

            for b in range(_NBUF):
                start_gather(b, b)

            def j_outer(oo, c2):
                o = oo * _NBUF
                for b in range(_NBUF):
                    j = o + b
                    wait_gather(j, b)

                    @pl.when(oo > 0)
                    def _():
                        wait_write(j - _NBUF, t, b)

                    transpose_block(b)
                    start_write(j, t, b)

                    @pl.when(o < _F - _NBUF)
                    def _():
                        start_gather(j + _NBUF, b)

                return c2

            lax.fori_loop(0, _F // _NBUF, j_outer, 0, unroll=False)

            for b in range(_NBUF):
                wait_write(_F - _NBUF + b, t, b)
            return carry

        lax.fori_loop(0, t_per_w, t_body, 0, unroll=False)

    return k


_W = 768
_NSTRIP = 999936 // _W
_TAIL = 1000000 - _NSTRIP * _W


def _make_table_transpose():
    info = plsc.get_sparse_core_info()
    nc, ns = info.num_cores, info.num_subcores
    nw = nc * ns
    base_strips = _NSTRIP // nw
    extra = _NSTRIP - base_strips * nw
    kmax = base_strips + 1

    mesh = plsc.VectorSubcoreMesh(core_axis_name="c", subcore_axis_name="s")

    @functools.partial(
        pl.kernel,
        mesh=mesh,
        compiler_params=pltpu.CompilerParams(needs_layout_passes=False),
        out_type=jax.ShapeDtypeStruct((250000, 128), jnp.float32),
        scratch_types=(
            [pltpu.VMEM((_D, _W), jnp.float32) for _ in range(2)]
            + [pltpu.VMEM((_W * _D // 128, 128), jnp.float32) for _ in range(2)]
            + [pltpu.SemaphoreType.DMA for _ in range(4)]
        ),
    )
    def k1(tt_hbm, tail_hbm, out_hbm, *bufs):
        sin = bufs[0:2]
        sout = bufs[2:4]
        gsem = bufs[4:6]
        wsem = bufs[6:8]

        wid = lax.axis_index("s") * nc + lax.axis_index("c")
        nmine = jnp.where(wid < extra, base_strips + 1, base_strips)
        lane_iota = lax.iota(jnp.int32, _L)

        @pl.when(wid == 0)
        def _():
            pltpu.sync_copy(tail_hbm, sout[0].at[pl.ds(0, _TAIL * _D // 128)])
            pltpu.sync_copy(sout[0].at[pl.ds(0, _TAIL * _D // 128)],
                            out_hbm.at[pl.ds(_NSTRIP * _W * _D // 128,
                                             _TAIL * _D // 128)])

        def strip_of(k):
            return wid + k * nw

        def start_read(k, b):
            pltpu.async_copy(
                tt_hbm.at[:, pl.ds(strip_of(k) * _W, _W)], sin[b], gsem[b])

        def wait_read(k, b):
            pltpu.make_async_copy(
                tt_hbm.at[:, pl.ds(strip_of(k) * _W, _W)], sin[b],
                gsem[b]).wait()

        def start_write(k, b):
            pltpu.async_copy(
                sout[b],
                out_hbm.at[pl.ds(strip_of(k) * (_W * _D // 128),
                                 _W * _D // 128)],
                wsem[b])

        def wait_write(k, b):
            pltpu.make_async_copy(
                sout[b],
                out_hbm.at[pl.ds(strip_of(k) * (_W * _D // 128),
                                 _W * _D // 128)],
                wsem[b]).wait()

        def transpose_strip(b):
            def d_body(d, c2):
                rot = lax.bitwise_and(lane_iota + d, _L - 1)
                for c0 in (0, 16):
                    for gq in range(3):
                        vals = []
                        for gg in range(_W // _L // 3):
                            g = gq * (_W // _L // 3) + gg
                            vals.append(plsc.load_gather(
                                sin[b], [c0 + lane_iota, rot + g * _L]))
                        for gg in range(_W // _L // 3):
                            g = gq * (_W // _L // 3) + gg
                            w = rot * _D + (g * _L * _D + c0) + lane_iota
                            plsc.store_scatter(
                                sout[b],
                                [lax.shift_right_logical(w, 7),
                                 lax.bitwise_and(w, 127)],
                                vals[gg])
                return c2

            lax.fori_loop(0, _L, d_body, 0, unroll=1)

        for b in range(2):
            start_read(b, b)

        def outer(kk, c2):
            for b in range(2):
                k = kk * 2 + b

                @pl.when(k < nmine)
                def _():
                    wait_read(k, b)

                    @pl.when(k >= 2)
                    def _():
                        wait_write(k - 2, b)

                    transpose_strip(b)
                    start_write(k, b)

                    @pl.when(k + 2 < nmine)
                    def _():
                        start_read(k + 2, b)

            return c2

        lax.fori_loop(0, (kmax + 1) // 2, outer, 0, unroll=False)

        for b in range(2):
            @pl.when(nmine - 2 + b >= 0)
            def _():
                wait_write(nmine - 2 + b, b)

    return k1


def kernel(x, table):
    b, f = x.shape
    xp = jnp.pad(x.astype(jnp.int32), ((0, 0), (0, _FP - f))).reshape(-1)
    tt = jnp.transpose(table)
    tail = table[_NSTRIP * _W:].reshape(_TAIL * _D // 128, 128)
    t128 = _make_table_transpose()(tt, tail)
    flat = _make_gather(b)(xp, t128.reshape(_NB, _D))
    n_tiles = b // _TI
    out5 = flat.reshape(f, 4, n_tiles, 8, _TI)
    return jnp.transpose(out5, (2, 4, 0, 1, 3)).reshape(b, f, _D)
